# baseline (speedup 1.0000x reference)
import jax
import jax.numpy as jnp
from jax.experimental import pallas as pl

BLK_S = 512


def _add_kernel(x_ref, pos_ref, o_ref):
    o_ref[...] = x_ref[...] + pos_ref[...][None, :, :]


def kernel(x, pos_table):
    batch, seq_len, embed = x.shape
    grid = (seq_len // BLK_S,)
    return pl.pallas_call(
        _add_kernel,
        grid=grid,
        in_specs=[
            pl.BlockSpec((batch, BLK_S, embed), lambda s: (0, s, 0)),
            pl.BlockSpec((BLK_S, embed), lambda s: (s, 0)),
        ],
        out_specs=pl.BlockSpec((batch, BLK_S, embed), lambda s: (0, s, 0)),
        out_shape=jax.ShapeDtypeStruct((batch, seq_len, embed), x.dtype),
    )(x, pos_table[:seq_len])
